# baseline (speedup 1.0000x reference)
import functools

import jax
import jax.numpy as jnp
from jax import lax
from jax.experimental import pallas as pl
from jax.experimental.pallas import tpu as pltpu
from jax.experimental.pallas import tpu_sc as plsc

NC = 2
NS = 16
NW = NC * NS


def _matmul_bias_body(x_ref, w_ref, b_ref, o_ref):
  o_ref[...] = (
      jnp.dot(x_ref[...], w_ref[...], preferred_element_type=jnp.float32)
      + b_ref[...]
  )


def _dense(x, w, b, block_rows):
  m, k = x.shape
  n = w.shape[1]
  assert m % block_rows == 0
  grid = (m // block_rows,)
  return pl.pallas_call(
      _matmul_bias_body,
      out_shape=jax.ShapeDtypeStruct((m, n), jnp.float32),
      grid=grid,
      in_specs=[
          pl.BlockSpec((block_rows, k), lambda i: (i, 0)),
          pl.BlockSpec((k, n), lambda i: (0, 0)),
          pl.BlockSpec((1, n), lambda i: (0, 0)),
      ],
      out_specs=pl.BlockSpec((block_rows, n), lambda i: (i, 0)),
  )(x, w, b)


def _bn(z, g_ref, b_ref):
  mu = jnp.mean(z, axis=0, keepdims=True)
  var = jnp.mean((z - mu) * (z - mu), axis=0, keepdims=True)
  return g_ref[...] * (z - mu) / jnp.sqrt(var + 1e-5) + b_ref[...]


def _mlp_body(h_ref, agg_ref, eps_ref, w1_ref, b1_ref, gm_ref, bm_ref,
              w2_ref, b2_ref, g_ref, b_ref, o_ref, *, final_relu):
  agg = agg_ref[0] + agg_ref[1]
  z = (1.0 + eps_ref[0, 0]) * h_ref[...] + agg
  z1 = jnp.dot(z, w1_ref[...], preferred_element_type=jnp.float32) + b1_ref[...]
  z1 = jnp.maximum(_bn(z1, gm_ref, bm_ref), 0.0)
  z2 = jnp.dot(z1, w2_ref[...], preferred_element_type=jnp.float32) + b2_ref[...]
  z2 = _bn(z2, g_ref, b_ref)
  if final_relu:
    z2 = jnp.maximum(z2, 0.0)
  o_ref[...] = z2


def _node_mlp(h, agg2, eps, w1, b1, gm, bm, w2, b2, g, b, final_relu):
  n, d = h.shape
  return pl.pallas_call(
      functools.partial(_mlp_body, final_relu=final_relu),
      out_shape=jax.ShapeDtypeStruct((n, d), jnp.float32),
  )(h, agg2, eps, w1, b1, gm, bm, w2, b2, g, b)


def _make_edge_pass(n_nodes, n_edges, d, chunk):
  epw = n_edges // NW
  nchunk = epw // chunk
  assert epw * NW == n_edges and nchunk * chunk == epw
  assert chunk % 8 == 0 and chunk <= 128
  assert nchunk >= 8
  zero_tiles = 10
  rows_per_tile = n_nodes // zero_tiles
  assert rows_per_tile * zero_tiles == n_nodes and rows_per_tile % 8 == 0

  mesh = plsc.VectorSubcoreMesh(core_axis_name="c", subcore_axis_name="s")

  @functools.partial(
      pl.kernel,
      out_type=jax.ShapeDtypeStruct((NC, n_nodes, d), jnp.float32),
      mesh=mesh,
      scratch_types=[
          [pltpu.VMEM((chunk,), jnp.int32)] * 4,
          [pltpu.VMEM((chunk,), jnp.int32)] * 4,
          [pltpu.VMEM((chunk, d), jnp.float32)] * 4,
          pltpu.VMEM_SHARED((n_nodes, d), jnp.float32),
          [pltpu.SemaphoreType.DMA] * 4,
          [pltpu.SemaphoreType.DMA] * 4,
          [pltpu.SemaphoreType.DMA] * 4,
          [pltpu.SemaphoreType.DMA] * 4,
      ],
  )
  def edge_pass(h_hbm, e_hbm, eidx_hbm, zero_hbm, out_hbm,
                src_v, dst_v, rows_v, agg_sh, isem, gsem, esem, ssem):
    cid = lax.axis_index("c")
    sid = lax.axis_index("s")
    wid = cid * NS + sid

    zrow = sid * rows_per_tile

    @pl.when(sid < zero_tiles)
    def _zero():
      pltpu.sync_copy(zero_hbm.at[pl.ds(zrow, rows_per_tile)],
                      agg_sh.at[pl.ds(zrow, rows_per_tile)])

    plsc.subcore_barrier()

    ebase = wid * epw

    def fire_idx(t, r):
      off = ebase + t * chunk
      pltpu.async_copy(eidx_hbm.at[pl.ds(off, chunk)], src_v[r], isem[r])
      pltpu.async_copy(eidx_hbm.at[pl.ds(n_edges + off, chunk)], dst_v[r],
                       isem[r])

    def fire_e(t, r):
      pltpu.async_copy(e_hbm.at[pl.ds(ebase + t * chunk, chunk)],
                       rows_v[r], esem[r])

    def fire_ga(r):
      pltpu.make_async_copy(eidx_hbm.at[pl.ds(0, chunk)], src_v[r],
                            isem[r]).wait()
      pltpu.make_async_copy(eidx_hbm.at[pl.ds(0, chunk)], dst_v[r],
                            isem[r]).wait()
      pltpu.make_async_copy(e_hbm.at[pl.ds(0, chunk)], rows_v[r],
                            esem[r]).wait()
      pltpu.async_copy(h_hbm.at[src_v[r]], rows_v[r], gsem[r], add=True)

    def compute(r):
      def row_body(i, c2):
        for u in range(4):
          for v in range(d // 16):
            s = pl.ds(v * 16, 16)
            rows_v[r][i * 4 + u, s] = jnp.maximum(rows_v[r][i * 4 + u, s],
                                                  0.0)
        return c2

      lax.fori_loop(0, chunk // 4, row_body, 0, unroll=False)

    def step(t, k):
      r = k % 4
      if not isinstance(t, int) or t >= 2:
        pltpu.make_async_copy(rows_v[(k + 2) % 4],
                              agg_sh.at[dst_v[(k + 2) % 4]],
                              ssem[(k + 2) % 4]).wait()
      if not isinstance(t, int) or t + 2 < nchunk:
        fire_idx(t + 2, (k + 2) % 4)
        fire_e(t + 2, (k + 2) % 4)
      if not isinstance(t, int) or t + 1 < nchunk:
        fire_ga((k + 1) % 4)
      pltpu.make_async_copy(h_hbm.at[src_v[r]], rows_v[r], gsem[r]).wait()
      compute(r)
      pltpu.async_copy(rows_v[r], agg_sh.at[dst_v[r]], ssem[r], add=True)

    fire_idx(0, 0)
    fire_e(0, 0)
    fire_idx(1, 1)
    fire_e(1, 1)
    fire_ga(0)
    step(0, 0)
    step(1, 1)

    nquad = (nchunk - 4) // 4

    def quad_body(i, carry):
      t = 2 + 4 * i
      step(t, 2)
      step(t + 1, 3)
      step(t + 2, 0)
      step(t + 3, 1)
      return carry

    lax.fori_loop(0, nquad, quad_body, 0, unroll=False)

    for t in range(2 + 4 * nquad, nchunk):
      step(t, t % 4)

    pltpu.make_async_copy(rows_v[(nchunk - 2) % 4],
                          agg_sh.at[dst_v[(nchunk - 2) % 4]],
                          ssem[(nchunk - 2) % 4]).wait()
    pltpu.make_async_copy(rows_v[(nchunk - 1) % 4],
                          agg_sh.at[dst_v[(nchunk - 1) % 4]],
                          ssem[(nchunk - 1) % 4]).wait()

    plsc.subcore_barrier()

    @pl.when(sid < zero_tiles)
    def _dump():
      pltpu.sync_copy(agg_sh.at[pl.ds(zrow, rows_per_tile)],
                      out_hbm.at[cid].at[pl.ds(zrow, rows_per_tile)])

  return edge_pass


def kernel(x, edge_index, edge_attr, batch, params):
  n, d = x.shape
  e_cnt = edge_index.shape[1]
  chunk = 80
  nchunk = e_cnt // (NW * chunk)
  eidx = edge_index.reshape(-1)
  ea = jnp.pad(edge_attr, ((0, 0), (0, 16 - edge_attr.shape[1])))
  zero_n = jnp.zeros((n, d), jnp.float32)

  edge_pass = _make_edge_pass(n, e_cnt, d, chunk=chunk)

  h = _dense(x, params['W_atom'], params['b_atom'].reshape(1, -1),
             block_rows=2000)
  num_layers = 2
  for l in range(num_layers):
    wb = jnp.pad(params['Wb%d' % l], ((0, 16 - params['Wb%d' % l].shape[0]),
                                      (0, 0)))
    e = _dense(ea, wb, params['bb%d' % l].reshape(1, -1), block_rows=8000)
    agg2 = edge_pass(h, e, eidx, zero_n)
    h = _node_mlp(
        h, agg2,
        params['eps%d' % l].reshape(1, 1),
        params['W1_%d' % l], params['b1_%d' % l].reshape(1, -1),
        params['gm%d' % l].reshape(1, -1), params['bm%d' % l].reshape(1, -1),
        params['W2_%d' % l], params['b2_%d' % l].reshape(1, -1),
        params['g%d' % l].reshape(1, -1), params['b%d' % l].reshape(1, -1),
        final_relu=(l < num_layers - 1),
    )
  return h


# speedup vs baseline: 1.0578x; 1.0578x over previous
import functools

import jax
import jax.numpy as jnp
from jax import lax
from jax.experimental import pallas as pl
from jax.experimental.pallas import tpu as pltpu
from jax.experimental.pallas import tpu_sc as plsc

NC = 2
NS = 16
NW = NC * NS


def _matmul_bias_body(x_ref, w_ref, b_ref, o_ref):
  o_ref[...] = (
      jnp.dot(x_ref[...], w_ref[...], preferred_element_type=jnp.float32)
      + b_ref[...]
  )


def _dense(x, w, b, block_rows):
  m, k = x.shape
  n = w.shape[1]
  assert m % block_rows == 0
  grid = (m // block_rows,)
  return pl.pallas_call(
      _matmul_bias_body,
      out_shape=jax.ShapeDtypeStruct((m, n), jnp.float32),
      grid=grid,
      in_specs=[
          pl.BlockSpec((block_rows, k), lambda i: (i, 0)),
          pl.BlockSpec((k, n), lambda i: (0, 0)),
          pl.BlockSpec((1, n), lambda i: (0, 0)),
      ],
      out_specs=pl.BlockSpec((block_rows, n), lambda i: (i, 0)),
  )(x, w, b)


def _bn(z, g_ref, b_ref):
  mu = jnp.mean(z, axis=0, keepdims=True)
  var = jnp.mean((z - mu) * (z - mu), axis=0, keepdims=True)
  return g_ref[...] * (z - mu) / jnp.sqrt(var + 1e-5) + b_ref[...]


def _mlp_body(h_ref, agg_ref, eps_ref, w1_ref, b1_ref, gm_ref, bm_ref,
              w2_ref, b2_ref, g_ref, b_ref, o_ref, *, final_relu):
  agg = agg_ref[0] + agg_ref[1]
  z = (1.0 + eps_ref[0, 0]) * h_ref[...] + agg
  z1 = jnp.dot(z, w1_ref[...], preferred_element_type=jnp.float32) + b1_ref[...]
  z1 = jnp.maximum(_bn(z1, gm_ref, bm_ref), 0.0)
  z2 = jnp.dot(z1, w2_ref[...], preferred_element_type=jnp.float32) + b2_ref[...]
  z2 = _bn(z2, g_ref, b_ref)
  if final_relu:
    z2 = jnp.maximum(z2, 0.0)
  o_ref[...] = z2


def _node_mlp(h, agg2, eps, w1, b1, gm, bm, w2, b2, g, b, final_relu):
  n, d = h.shape
  return pl.pallas_call(
      functools.partial(_mlp_body, final_relu=final_relu),
      out_shape=jax.ShapeDtypeStruct((n, d), jnp.float32),
  )(h, agg2, eps, w1, b1, gm, bm, w2, b2, g, b)


def _make_edge_pass(n_nodes, n_edges, d, chunk):
  epw = n_edges // NW
  nchunk = epw // chunk
  assert epw * NW == n_edges and nchunk * chunk == epw
  assert chunk % 8 == 0 and chunk <= 128
  assert nchunk >= 8
  zero_tiles = 10
  rows_per_tile = n_nodes // zero_tiles
  assert rows_per_tile * zero_tiles == n_nodes and rows_per_tile % 8 == 0

  mesh = plsc.VectorSubcoreMesh(core_axis_name="c", subcore_axis_name="s")

  @functools.partial(
      pl.kernel,
      out_type=jax.ShapeDtypeStruct((NC, n_nodes, d), jnp.float32),
      mesh=mesh,
      scratch_types=[
          [pltpu.VMEM((chunk,), jnp.int32)] * 4,
          [pltpu.VMEM((chunk,), jnp.int32)] * 4,
          [pltpu.VMEM((chunk, d), jnp.float32)] * 4,
          pltpu.VMEM_SHARED((n_nodes, d), jnp.float32),
          [pltpu.SemaphoreType.DMA] * 4,
          [pltpu.SemaphoreType.DMA] * 4,
          [pltpu.SemaphoreType.DMA] * 4,
          [pltpu.SemaphoreType.DMA] * 4,
      ],
  )
  def edge_pass(h_hbm, e_hbm, eidx_hbm, zero_hbm, out_hbm,
                src_v, dst_v, rows_v, agg_sh, isem, gsem, esem, ssem):
    cid = lax.axis_index("c")
    sid = lax.axis_index("s")
    wid = cid * NS + sid

    zrow = sid * rows_per_tile

    @pl.when(sid < zero_tiles)
    def _zero():
      pltpu.sync_copy(zero_hbm.at[pl.ds(zrow, rows_per_tile)],
                      agg_sh.at[pl.ds(zrow, rows_per_tile)])

    plsc.subcore_barrier()

    ebase = wid * epw

    def fire_idx(t, r):
      off = ebase + t * chunk
      pltpu.async_copy(eidx_hbm.at[pl.ds(off, chunk)], src_v[r], isem[r])
      pltpu.async_copy(eidx_hbm.at[pl.ds(n_edges + off, chunk)], dst_v[r],
                       isem[r])

    def fire_e(t, r):
      pltpu.async_copy(e_hbm.at[pl.ds(ebase + t * chunk, chunk)],
                       rows_v[r], esem[r])

    def fire_ga(r):
      pltpu.make_async_copy(eidx_hbm.at[pl.ds(0, chunk)], src_v[r],
                            isem[r]).wait()
      pltpu.make_async_copy(eidx_hbm.at[pl.ds(0, chunk)], dst_v[r],
                            isem[r]).wait()
      pltpu.make_async_copy(e_hbm.at[pl.ds(0, chunk)], rows_v[r],
                            esem[r]).wait()
      pltpu.async_copy(h_hbm.at[src_v[r]], rows_v[r], gsem[r], add=True)

    def compute(r):
      def row_body(i, c2):
        for u in range(4):
          for v in range(d // 16):
            s = pl.ds(v * 16, 16)
            rows_v[r][i * 4 + u, s] = jnp.maximum(rows_v[r][i * 4 + u, s],
                                                  0.0)
        return c2

      lax.fori_loop(0, chunk // 4, row_body, 0, unroll=False)

    def step(t, k):
      r = k % 4
      if not isinstance(t, int) or t >= 2:
        pltpu.make_async_copy(rows_v[(k + 2) % 4],
                              agg_sh.at[dst_v[(k + 2) % 4]],
                              ssem[(k + 2) % 4]).wait()
      if not isinstance(t, int) or t + 2 < nchunk:
        fire_idx(t + 2, (k + 2) % 4)
        fire_e(t + 2, (k + 2) % 4)
      if not isinstance(t, int) or t + 1 < nchunk:
        fire_ga((k + 1) % 4)
      pltpu.make_async_copy(h_hbm.at[src_v[r]], rows_v[r], gsem[r]).wait()
      compute(r)
      pltpu.async_copy(rows_v[r], agg_sh.at[dst_v[r]], ssem[r], add=True)

    fire_idx(0, 0)
    fire_e(0, 0)
    fire_idx(1, 1)
    fire_e(1, 1)
    fire_ga(0)
    step(0, 0)
    step(1, 1)

    nquad = (nchunk - 4) // 4

    def quad_body(i, carry):
      t = 2 + 4 * i
      step(t, 2)
      step(t + 1, 3)
      step(t + 2, 0)
      step(t + 3, 1)
      return carry

    lax.fori_loop(0, nquad, quad_body, 0, unroll=False)

    for t in range(2 + 4 * nquad, nchunk):
      step(t, t % 4)

    pltpu.make_async_copy(rows_v[(nchunk - 2) % 4],
                          agg_sh.at[dst_v[(nchunk - 2) % 4]],
                          ssem[(nchunk - 2) % 4]).wait()
    pltpu.make_async_copy(rows_v[(nchunk - 1) % 4],
                          agg_sh.at[dst_v[(nchunk - 1) % 4]],
                          ssem[(nchunk - 1) % 4]).wait()

    plsc.subcore_barrier()

    @pl.when(sid < zero_tiles)
    def _dump():
      pltpu.sync_copy(agg_sh.at[pl.ds(zrow, rows_per_tile)],
                      out_hbm.at[cid].at[pl.ds(zrow, rows_per_tile)])

  return edge_pass


def kernel(x, edge_index, edge_attr, batch, params):
  n, d = x.shape
  e_cnt = edge_index.shape[1]
  chunk = 80
  nchunk = e_cnt // (NW * chunk)
  eidx = edge_index.reshape(-1)
  zero_n = jnp.zeros((n, d), jnp.float32)

  edge_pass = _make_edge_pass(n, e_cnt, d, chunk=chunk)

  h = _dense(x, params['W_atom'], params['b_atom'].reshape(1, -1),
             block_rows=2000)
  num_layers = 2
  for l in range(num_layers):
    e = _dense(edge_attr, params['Wb%d' % l],
               params['bb%d' % l].reshape(1, -1), block_rows=8000)
    agg2 = edge_pass(h, e, eidx, zero_n)
    h = _node_mlp(
        h, agg2,
        params['eps%d' % l].reshape(1, 1),
        params['W1_%d' % l], params['b1_%d' % l].reshape(1, -1),
        params['gm%d' % l].reshape(1, -1), params['bm%d' % l].reshape(1, -1),
        params['W2_%d' % l], params['b2_%d' % l].reshape(1, -1),
        params['g%d' % l].reshape(1, -1), params['b%d' % l].reshape(1, -1),
        final_relu=(l < num_layers - 1),
    )
  return h
